# baseline (speedup 1.0000x reference)
import functools

import jax
import jax.numpy as jnp
from jax import lax
from jax.experimental import pallas as pl
from jax.experimental.pallas import tpu as pltpu
from jax.experimental.pallas import tpu_sc as plsc

EMB = 32
IDX_CHUNK = 128


def _sc_gather(uidx2d, iidx2d, t_ug, t_ig, t_um, t_im, batch):
    info = plsc.get_sparse_core_info()
    nc, ns = info.num_cores, info.num_subcores
    nw = nc * ns
    rows_per_w = batch // nw
    chunks = rows_per_w // IDX_CHUNK
    mesh = plsc.VectorSubcoreMesh(core_axis_name="c", subcore_axis_name="s")

    @functools.partial(
        pl.kernel,
        mesh=mesh,
        out_type=[jax.ShapeDtypeStruct((batch, EMB), jnp.float32)] * 4,
        scratch_types=[
            pltpu.VMEM((chunks, IDX_CHUNK), jnp.int32),
            pltpu.VMEM((chunks, IDX_CHUNK), jnp.int32),
            pltpu.VMEM((rows_per_w, EMB), jnp.float32),
            pltpu.VMEM((rows_per_w, EMB), jnp.float32),
            pltpu.VMEM((rows_per_w, EMB), jnp.float32),
            pltpu.VMEM((rows_per_w, EMB), jnp.float32),
            pltpu.SemaphoreType.DMA,
        ],
        compiler_params=pltpu.CompilerParams(use_tc_tiling_on_sc=False),
    )
    def k(uidx_hbm, iidx_hbm, ug_hbm, ig_hbm, um_hbm, im_hbm,
          oug, oig, oum, oim, uv, iv, rug, rig, rum, rim, sem):
        wid = lax.axis_index("s") * nc + lax.axis_index("c")
        crow = wid * chunks
        pltpu.sync_copy(uidx_hbm.at[pl.ds(crow, chunks)], uv)
        pltpu.sync_copy(iidx_hbm.at[pl.ds(crow, chunks)], iv)
        handles = []
        for j in range(chunks):
            for tbl, dst, idx in ((ug_hbm, rug, uv), (ig_hbm, rig, iv),
                                  (um_hbm, rum, uv), (im_hbm, rim, iv)):
                handles.append(pltpu.async_copy(
                    tbl.at[idx.at[j]],
                    dst.at[pl.ds(j * IDX_CHUNK, IDX_CHUNK)], sem))
        for h in handles:
            h.wait()
        base = wid * rows_per_w
        for dst_hbm, src in ((oug, rug), (oig, rig), (oum, rum), (oim, rim)):
            pltpu.sync_copy(src, dst_hbm.at[pl.ds(base, rows_per_w)])

    return k(uidx2d, iidx2d, t_ug, t_ig, t_um, t_im)


def _tc_dense(gu, gi, mu, mi, w1u, w1i, b1, w2, b2, w3, b3, w4, b4,
              wpg, wph, bp):
    batch = gu.shape[0]

    def body(gu_ref, gi_ref, mu_ref, mi_ref, w1u_ref, w1i_ref, b1_ref,
             w2_ref, b2_ref, w3_ref, b3_ref, w4_ref, b4_ref,
             wpg_ref, wph_ref, bp_ref, out_ref):
        dg = lambda x, w: lax.dot_general(
            x, w, (((1,), (1,)), ((), ())),
            preferred_element_type=jnp.float32)
        h = jnp.maximum(dg(mu_ref[...], w1u_ref[...])
                        + dg(mi_ref[...], w1i_ref[...]) + b1_ref[...], 0.0)
        h = jnp.maximum(dg(h, w2_ref[...]) + b2_ref[...], 0.0)
        h = jnp.maximum(dg(h, w3_ref[...]) + b3_ref[...], 0.0)
        h = jnp.maximum(dg(h, w4_ref[...]) + b4_ref[...], 0.0)
        g = gu_ref[...] * gi_ref[...]
        pred = (jnp.sum(g * wpg_ref[...], axis=1)
                + jnp.sum(h * wph_ref[...], axis=1) + bp_ref[0, 0])
        out_ref[...] = jax.nn.sigmoid(pred)

    return pl.pallas_call(
        body,
        out_shape=jax.ShapeDtypeStruct((batch,), jnp.float32),
    )(gu, gi, mu, mi, w1u, w1i, b1, w2, b2, w3, b3, w4, b4, wpg, wph, bp)


def kernel(user_indices, item_indices, emb_user_gmf, emb_item_gmf,
           emb_user_mlp, emb_item_mlp, W1, b1, W2, b2, W3, b3, W4, b4,
           Wp, bp):
    batch = user_indices.shape[0]
    uidx2d = user_indices.astype(jnp.int32).reshape(batch // IDX_CHUNK,
                                                    IDX_CHUNK)
    iidx2d = item_indices.astype(jnp.int32).reshape(batch // IDX_CHUNK,
                                                    IDX_CHUNK)
    gu, gi, mu, mi = _sc_gather(uidx2d, iidx2d, emb_user_gmf, emb_item_gmf,
                                emb_user_mlp, emb_item_mlp, batch)
    return _tc_dense(
        gu, gi, mu, mi,
        W1[:, :EMB], W1[:, EMB:], b1.reshape(1, -1),
        W2, b2.reshape(1, -1), W3, b3.reshape(1, -1),
        W4, b4.reshape(1, -1),
        Wp[:, :EMB], Wp[:, EMB:], bp.reshape(1, 1))
